# Initial kernel scaffold; baseline (speedup 1.0000x reference)
import jax
import jax.numpy as jnp
from jax.experimental import pallas as pl


def kernel(x, edge_index, adj_values, W):
    raise NotImplementedError("write your pallas kernel here")


# speedup vs baseline: 4.2859x; 4.2859x over previous
import functools

import jax
import jax.numpy as jnp
from jax import lax
from jax.experimental import pallas as pl
from jax.experimental.pallas import tpu as pltpu
from jax.experimental.pallas import tpu_sc as plsc

N = 10000
D_IN = 128
D_OUT = 128
D_HALF = D_OUT // 2
NSC = 2
NSUB = 16
CHUNK = 128
ROWS_PER_SUB = N // NSUB
RELU_BLK = 125


def _matmul_body(x_ref, w_ref, o_ref):
    o_ref[0] = jnp.dot(x_ref[...], w_ref[0], preferred_element_type=jnp.float32)


def _matmul_split(x, w_split, row_blk):
    n = x.shape[0]
    grid = (NSC, n // row_blk)
    return pl.pallas_call(
        _matmul_body,
        grid=grid,
        in_specs=[
            pl.BlockSpec((row_blk, D_IN), lambda c, i: (i, 0)),
            pl.BlockSpec((1, D_IN, D_HALF), lambda c, i: (c, 0, 0)),
        ],
        out_specs=pl.BlockSpec((1, row_blk, D_HALF), lambda c, i: (c, i, 0)),
        out_shape=jax.ShapeDtypeStruct((NSC, n, D_HALF), jnp.float32),
    )(x, w_split)


def _make_sc_kernel(n_chunks):
    mesh = plsc.VectorSubcoreMesh(core_axis_name="c", subcore_axis_name="s")

    @functools.partial(
        pl.kernel,
        mesh=mesh,
        out_type=jax.ShapeDtypeStruct((NSC, N, D_HALF), jnp.float32),
        compiler_params=pltpu.CompilerParams(
            use_tc_tiling_on_sc=False, needs_layout_passes=False),
        scratch_types=[
            pltpu.VMEM((n_chunks, CHUNK), jnp.int32),
            pltpu.VMEM((n_chunks, CHUNK), jnp.int32),
            pltpu.VMEM((n_chunks, CHUNK), jnp.float32),
            pltpu.VMEM((CHUNK, D_HALF), jnp.float32),
            pltpu.VMEM_SHARED((N, D_HALF), jnp.float32),
            pltpu.SemaphoreType.DMA,
        ],
    )
    def spmm(h_hbm, src_hbm, dst_hbm, val_hbm, out_hbm,
             src_v, dst_v, val_v, rows_v, acc, sem):
        c = lax.axis_index("c")
        s = lax.axis_index("s")

        pltpu.sync_copy(src_hbm.at[s], src_v)
        pltpu.sync_copy(dst_hbm.at[s], dst_v)
        pltpu.sync_copy(val_hbm.at[s], val_v)

        off = c * N

        def adjust(i, _):
            for k in range(CHUNK // 16):
                sl = pl.ds(16 * k, 16)
                src_v[i, sl] = src_v[i, sl] + off
            return ()

        lax.fori_loop(0, n_chunks, adjust, ())

        def zero_rows(i, _):
            for k in range(D_HALF // 16):
                rows_v[i, pl.ds(16 * k, 16)] = jnp.zeros((16,), jnp.float32)
            return ()

        lax.fori_loop(0, CHUNK, zero_rows, ())
        for b in range(ROWS_PER_SUB // RELU_BLK):
            pltpu.sync_copy(
                rows_v.at[pl.ds(0, RELU_BLK)],
                acc.at[pl.ds(s * ROWS_PER_SUB + b * RELU_BLK, RELU_BLK)],
            )
        plsc.subcore_barrier()

        def chunk_body(j, _):
            pltpu.async_copy(h_hbm.at[src_v.at[j]], rows_v, sem).wait()

            jidx = jnp.full((16,), j, jnp.int32)

            def scale(r, _):
                v = plsc.load_gather(val_v, [jidx, jnp.full((16,), r, jnp.int32)])
                for k in range(D_HALF // 16):
                    sl = pl.ds(16 * k, 16)
                    rows_v[r, sl] = rows_v[r, sl] * v
                return ()

            lax.fori_loop(0, CHUNK, scale, ())
            pltpu.sync_copy(rows_v, acc.at[dst_v.at[j]], add=True)
            return ()

        lax.fori_loop(0, n_chunks, chunk_body, ())
        plsc.subcore_barrier()

        for b in range(ROWS_PER_SUB // RELU_BLK):
            row0 = s * ROWS_PER_SUB + b * RELU_BLK
            pltpu.sync_copy(acc.at[pl.ds(row0, RELU_BLK)],
                            rows_v.at[pl.ds(0, RELU_BLK)])

            def relu(r, _):
                for k in range(D_HALF // 16):
                    sl = pl.ds(16 * k, 16)
                    rows_v[r, sl] = jnp.maximum(rows_v[r, sl], 0.0)
                return ()

            lax.fori_loop(0, RELU_BLK, relu, ())
            pltpu.sync_copy(rows_v.at[pl.ds(0, RELU_BLK)],
                            out_hbm.at[c, pl.ds(row0, RELU_BLK)])

    return spmm


def kernel(x, edge_index, adj_values, W):
    e = edge_index.shape[1]
    n_chunks = -(-e // (NSUB * CHUNK))
    e_pad = NSUB * n_chunks * CHUNK
    pad = e_pad - e

    src = jnp.concatenate([edge_index[0], jnp.zeros((pad,), jnp.int32)])
    dst = jnp.concatenate([edge_index[1], jnp.zeros((pad,), jnp.int32)])
    val = jnp.concatenate([adj_values, jnp.zeros((pad,), jnp.float32)])
    src = src.reshape(NSUB, n_chunks, CHUNK)
    dst = dst.reshape(NSUB, n_chunks, CHUNK)
    val = val.reshape(NSUB, n_chunks, CHUNK)

    w_split = W.reshape(D_IN, NSC, D_HALF).transpose(1, 0, 2)
    h_split = _matmul_split(x, w_split, row_blk=1000)
    h_flat = h_split.reshape(NSC * N, D_HALF)

    out2 = _make_sc_kernel(n_chunks)(h_flat, src, dst, val)
    return out2.transpose(1, 0, 2).reshape(N, D_OUT)


# speedup vs baseline: 6.6604x; 1.5540x over previous
import functools

import jax
import jax.numpy as jnp
from jax import lax
from jax.experimental import pallas as pl
from jax.experimental.pallas import tpu as pltpu
from jax.experimental.pallas import tpu_sc as plsc

N = 10000
D_IN = 128
D_OUT = 128
D_HALF = D_OUT // 2
NSC = 2
NSUB = 16
CHUNK = 128
ROWS_PER_SUB = N // NSUB
RELU_BLK = 125


def _matmul_body(x_ref, w_ref, o_ref):
    o_ref[0] = jnp.dot(x_ref[...], w_ref[0], preferred_element_type=jnp.float32)


def _matmul_split(x, w_split, row_blk):
    n = x.shape[0]
    grid = (NSC, n // row_blk)
    return pl.pallas_call(
        _matmul_body,
        grid=grid,
        in_specs=[
            pl.BlockSpec((row_blk, D_IN), lambda c, i: (i, 0)),
            pl.BlockSpec((1, D_IN, D_HALF), lambda c, i: (c, 0, 0)),
        ],
        out_specs=pl.BlockSpec((1, row_blk, D_HALF), lambda c, i: (c, i, 0)),
        out_shape=jax.ShapeDtypeStruct((NSC, n, D_HALF), jnp.float32),
    )(x, w_split)


NBUF = 2


def _make_sc_kernel(n_chunks):
    assert n_chunks % NBUF == 0
    mesh = plsc.VectorSubcoreMesh(core_axis_name="c", subcore_axis_name="s")

    @functools.partial(
        pl.kernel,
        mesh=mesh,
        out_type=jax.ShapeDtypeStruct((NSC, N, D_HALF), jnp.float32),
        compiler_params=pltpu.CompilerParams(
            use_tc_tiling_on_sc=False, needs_layout_passes=False),
        scratch_types=[
            pltpu.VMEM((n_chunks, CHUNK), jnp.int32),
            pltpu.VMEM((n_chunks, CHUNK), jnp.int32),
            pltpu.VMEM((n_chunks, CHUNK), jnp.float32),
            pltpu.VMEM((NBUF, CHUNK, D_HALF), jnp.float32),
            pltpu.VMEM_SHARED((N, D_HALF), jnp.float32),
            pltpu.SemaphoreType.DMA((NBUF,)),
            pltpu.SemaphoreType.DMA((NBUF,)),
        ],
    )
    def spmm(h_hbm, src_hbm, dst_hbm, val_hbm, out_hbm,
             src_v, dst_v, val_v, rows_v, acc, gsem, ssem):
        c = lax.axis_index("c")
        s = lax.axis_index("s")

        pltpu.sync_copy(src_hbm.at[s], src_v)
        pltpu.sync_copy(dst_hbm.at[s], dst_v)
        pltpu.sync_copy(val_hbm.at[s], val_v)

        off = c * N

        @plsc.parallel_loop(0, n_chunks, unroll=4)
        def _(i):
            for k in range(CHUNK // 16):
                sl = pl.ds(16 * k, 16)
                src_v[i, sl] = src_v[i, sl] + off

        @plsc.parallel_loop(0, CHUNK, unroll=4)
        def _(i):
            for k in range(D_HALF // 16):
                rows_v[0, i, pl.ds(16 * k, 16)] = jnp.zeros((16,), jnp.float32)

        for b in range(ROWS_PER_SUB // RELU_BLK):
            pltpu.sync_copy(
                rows_v.at[0, pl.ds(0, RELU_BLK)],
                acc.at[pl.ds(s * ROWS_PER_SUB + b * RELU_BLK, RELU_BLK)],
            )
        plsc.subcore_barrier()

        def start_gather(j, b):
            pltpu.async_copy(h_hbm.at[src_v.at[j]], rows_v.at[b], gsem.at[b])

        for b in range(NBUF):
            start_gather(b, b)

        def process(j, b):
            pltpu.make_async_copy(h_hbm.at[src_v.at[j]], rows_v.at[b],
                                  gsem.at[b]).wait()

            jidx = jnp.full((16,), j, jnp.int32)

            @plsc.parallel_loop(0, CHUNK, unroll=4)
            def _(r):
                v = plsc.load_gather(val_v, [jidx, jnp.full((16,), r, jnp.int32)])
                for k in range(D_HALF // 16):
                    sl = pl.ds(16 * k, 16)
                    rows_v[b, r, sl] = rows_v[b, r, sl] * v

            pltpu.async_copy(rows_v.at[b], acc.at[dst_v.at[j]], ssem.at[b],
                             add=True)

        def ring_body(g, _):
            for b in range(NBUF):
                j = g * NBUF + b
                process(j, b)

                @pl.when(j + NBUF < n_chunks)
                def _():
                    pltpu.make_async_copy(rows_v.at[b], acc.at[dst_v.at[j]],
                                          ssem.at[b]).wait()
                    start_gather(j + NBUF, b)
            return ()

        lax.fori_loop(0, n_chunks // NBUF, ring_body, ())

        for b in range(NBUF):
            j = n_chunks - NBUF + b
            pltpu.make_async_copy(rows_v.at[b], acc.at[dst_v.at[j]],
                                  ssem.at[b]).wait()
        plsc.subcore_barrier()

        for b in range(ROWS_PER_SUB // RELU_BLK):
            row0 = s * ROWS_PER_SUB + b * RELU_BLK
            buf = b % NBUF
            pltpu.sync_copy(acc.at[pl.ds(row0, RELU_BLK)],
                            rows_v.at[buf, pl.ds(0, RELU_BLK)])

            @plsc.parallel_loop(0, RELU_BLK, unroll=4)
            def _(r):
                for k in range(D_HALF // 16):
                    sl = pl.ds(16 * k, 16)
                    rows_v[buf, r, sl] = jnp.maximum(rows_v[buf, r, sl], 0.0)

            pltpu.sync_copy(rows_v.at[buf, pl.ds(0, RELU_BLK)],
                            out_hbm.at[c, pl.ds(row0, RELU_BLK)])

    return spmm


def kernel(x, edge_index, adj_values, W):
    e = edge_index.shape[1]
    n_chunks = -(-e // (NSUB * CHUNK))
    n_chunks = -(-n_chunks // NBUF) * NBUF
    e_pad = NSUB * n_chunks * CHUNK
    pad = e_pad - e

    src = jnp.concatenate([edge_index[0], jnp.zeros((pad,), jnp.int32)])
    dst = jnp.concatenate([edge_index[1], jnp.zeros((pad,), jnp.int32)])
    val = jnp.concatenate([adj_values, jnp.zeros((pad,), jnp.float32)])
    src = src.reshape(NSUB, n_chunks, CHUNK)
    dst = dst.reshape(NSUB, n_chunks, CHUNK)
    val = val.reshape(NSUB, n_chunks, CHUNK)

    w_split = W.reshape(D_IN, NSC, D_HALF).transpose(1, 0, 2)
    h_split = _matmul_split(x, w_split, row_blk=1000)
    h_flat = h_split.reshape(NSC * N, D_HALF)

    out2 = _make_sc_kernel(n_chunks)(h_flat, src, dst, val)
    return out2.transpose(1, 0, 2).reshape(N, D_OUT)
